# baseline (speedup 1.0000x reference)
import functools

import jax
import jax.numpy as jnp
from jax import lax
from jax.experimental import pallas as pl
from jax.experimental.pallas import tpu as pltpu
from jax.experimental.pallas import tpu_sc as plsc

N = 10000
E = 160000
D = 300
DE = 16
H = 600
G = 256
L = 5

HALF = 160
DP = 2 * HALF
HP = 640
NPAD = 10240

EB = 40
NBLK = 250
_CHUNK = 25
_NCH = NBLK // _CHUNK
TILE_E = EB * NBLK
NTILES = 16
_ROWS_PER_TILE = N // NTILES


_EBLK = 640


def _edge_encode_body(ea_ref, w_ref, b_ref, out_ref):
    ea = ea_ref[...]
    for s in range(2):
        out_ref[s] = (
            jnp.dot(ea, w_ref[s], preferred_element_type=jnp.float32)
            + b_ref[s]
        ).astype(jnp.bfloat16)


def _edge_encode(edge_attr, w2, b2):
    return pl.pallas_call(
        _edge_encode_body,
        grid=(E // _EBLK,),
        in_specs=[
            pl.BlockSpec((_EBLK, DE), lambda i: (i, 0)),
            pl.BlockSpec((2, DE, HALF), lambda i: (0, 0, 0)),
            pl.BlockSpec((2, 1, HALF), lambda i: (0, 0, 0)),
        ],
        out_specs=pl.BlockSpec((2, _EBLK, HALF), lambda i: (0, i, 0)),
        out_shape=jax.ShapeDtypeStruct((2, E, HALF), jnp.bfloat16),
    )(edge_attr, w2, b2)


def _sc_body(h_hbm, e_hbm, src_hbm, dst_hbm, zeros_hbm, agg_hbm,
             src_v, dst_v, h_a, h_b, e_a, e_b, agg_sh,
             sem_ha, sem_hb, sem_ea, sem_eb, sem_s):
    cid = lax.axis_index("c")
    sid = lax.axis_index("s")
    hsrc = h_hbm.at[cid]
    eslab = e_hbm.at[cid]

    row0 = sid * _ROWS_PER_TILE
    pltpu.sync_copy(zeros_hbm, agg_sh.at[pl.ds(row0, _ROWS_PER_TILE)])
    plsc.subcore_barrier()

    def _issue(g, j, hbuf, ebuf, sem_h, sem_e):
        pltpu.async_copy(hsrc.at[src_v.at[j]], hbuf, sem_h)
        pltpu.async_copy(
            eslab.at[pl.ds(sid * TILE_E + g * EB, EB)], ebuf, sem_e)

    def _work(j, hbuf, ebuf, sem_h, sem_e):
        pltpu.make_async_copy(hsrc.at[src_v.at[j]], hbuf, sem_h).wait()
        pltpu.make_async_copy(eslab.at[pl.ds(0, EB)], ebuf, sem_e).wait()

        def _row(b, _):
            for k2 in range(HALF // 32):
                ev = ebuf[b, pl.ds(k2 * 32, 32)]
                e0, e1 = plsc.unpack(ev, format=plsc.PackFormat.INTERLEAVED)
                s0 = pl.ds(k2 * 32, 16)
                s1 = pl.ds(k2 * 32 + 16, 16)
                hbuf[b, s0] = jnp.maximum(hbuf[b, s0] + e0, 0.0)
                hbuf[b, s1] = jnp.maximum(hbuf[b, s1] + e1, 0.0)
            return 0

        lax.fori_loop(0, EB, _row, 0)
        pltpu.async_copy(hbuf, agg_sh.at[dst_v.at[j]], sem_s, add=True)

    def _chunk(c, _):
        @pl.when(c > 0)
        def _():
            pltpu.make_async_copy(h_a, agg_sh.at[dst_v.at[0]], sem_s).wait()

        pltpu.sync_copy(src_hbm.at[sid].at[pl.ds(c * _CHUNK, _CHUNK)], src_v)
        pltpu.sync_copy(dst_hbm.at[sid].at[pl.ds(c * _CHUNK, _CHUNK)], dst_v)

        @pl.when(c % 2 == 0)
        def _():
            _issue(c * _CHUNK, 0, h_a, e_a, sem_ha, sem_ea)

        @pl.when(c % 2 == 1)
        def _():
            _issue(c * _CHUNK, 0, h_b, e_b, sem_hb, sem_eb)

        def _blk(j, _):
            g = c * _CHUNK + j
            par = (c + j) % 2

            @pl.when(j > 0)
            def _():
                pltpu.make_async_copy(h_a, agg_sh.at[dst_v.at[0]],
                                      sem_s).wait()

            @pl.when(jnp.logical_and(j < _CHUNK - 1, par == 0))
            def _():
                _issue(g + 1, j + 1, h_b, e_b, sem_hb, sem_eb)

            @pl.when(jnp.logical_and(j < _CHUNK - 1, par == 1))
            def _():
                _issue(g + 1, j + 1, h_a, e_a, sem_ha, sem_ea)

            @pl.when(par == 0)
            def _():
                _work(j, h_a, e_a, sem_ha, sem_ea)

            @pl.when(par == 1)
            def _():
                _work(j, h_b, e_b, sem_hb, sem_eb)

            return 0

        lax.fori_loop(0, _CHUNK, _blk, 0)
        return 0

    lax.fori_loop(0, _NCH, _chunk, 0)
    pltpu.make_async_copy(h_a, agg_sh.at[dst_v.at[0]], sem_s).wait()
    plsc.subcore_barrier()

    pltpu.sync_copy(agg_sh.at[pl.ds(row0, _ROWS_PER_TILE)],
                    agg_hbm.at[cid].at[pl.ds(row0, _ROWS_PER_TILE)])

    @pl.when(sid == 0)
    def _():
        pltpu.sync_copy(zeros_hbm.at[pl.ds(0, NPAD - N)],
                        agg_hbm.at[cid].at[pl.ds(N, NPAD - N)])


def _sc_message_agg(h, e_l, srcr, dstr, zeros):
    mesh = plsc.VectorSubcoreMesh(core_axis_name="c", subcore_axis_name="s",
                                  num_cores=2, num_subcores=NTILES)
    k = pl.kernel(
        _sc_body,
        out_type=jax.ShapeDtypeStruct((2, NPAD, HALF), jnp.float32),
        mesh=mesh,
        scratch_types=[
            pltpu.VMEM((_CHUNK, EB), jnp.int32),
            pltpu.VMEM((_CHUNK, EB), jnp.int32),
            pltpu.VMEM((EB, HALF), jnp.float32),
            pltpu.VMEM((EB, HALF), jnp.float32),
            pltpu.VMEM((EB, HALF), jnp.bfloat16),
            pltpu.VMEM((EB, HALF), jnp.bfloat16),
            pltpu.VMEM_SHARED((N, HALF), jnp.float32),
            pltpu.SemaphoreType.DMA,
            pltpu.SemaphoreType.DMA,
            pltpu.SemaphoreType.DMA,
            pltpu.SemaphoreType.DMA,
            pltpu.SemaphoreType.DMA,
        ],
        compiler_params=pltpu.CompilerParams(use_tc_tiling_on_sc=False,
                                             needs_layout_passes=False),
    )
    return k(h, e_l, srcr, dstr, zeros)


_NBLK_ROWS = 512


def _mlp_body(relu_out, h_ref, a_ref,
              w1a_ref, w1b_ref, b1_ref, w2a_ref, w2b_ref, b2a_ref, b2b_ref,
              o_ref):
    z0 = h_ref[0] + a_ref[0]
    z1 = h_ref[1] + a_ref[1]
    t = (jnp.dot(z0, w1a_ref[...], preferred_element_type=jnp.float32)
         + jnp.dot(z1, w1b_ref[...], preferred_element_type=jnp.float32)
         + b1_ref[...])
    t = jnp.maximum(t, 0.0)
    o0 = jnp.dot(t, w2a_ref[...], preferred_element_type=jnp.float32) + b2a_ref[...]
    o1 = jnp.dot(t, w2b_ref[...], preferred_element_type=jnp.float32) + b2b_ref[...]
    if relu_out:
        o0 = jnp.maximum(o0, 0.0)
        o1 = jnp.maximum(o1, 0.0)
    o_ref[0] = o0
    o_ref[1] = o1


def _mlp(relu_out, h, a, w1a, w1b, b1, w2a, w2b, b2a, b2b):
    nb = NPAD // _NBLK_ROWS
    row = lambda i: (0, i, 0)
    rep = lambda i: (0, 0)
    return pl.pallas_call(
        functools.partial(_mlp_body, relu_out),
        grid=(nb,),
        in_specs=[
            pl.BlockSpec((2, _NBLK_ROWS, HALF), row),
            pl.BlockSpec((2, _NBLK_ROWS, HALF), row),
            pl.BlockSpec((HALF, HP), rep),
            pl.BlockSpec((HALF, HP), rep),
            pl.BlockSpec((1, HP), rep),
            pl.BlockSpec((HP, HALF), rep),
            pl.BlockSpec((HP, HALF), rep),
            pl.BlockSpec((1, HALF), rep),
            pl.BlockSpec((1, HALF), rep),
        ],
        out_specs=pl.BlockSpec((2, _NBLK_ROWS, HALF), row),
        out_shape=jax.ShapeDtypeStruct((2, NPAD, HALF), jnp.float32),
    )(h, a, w1a, w1b, b1, w2a, w2b, b2a, b2b)


def _proj_pool_body(h_ref, p1a_ref, p1b_ref, pb1_ref, p2_ref,
                    pb2_ref, bat_ref, gf_ref, acc_ref, cnt_ref):
    i = pl.program_id(0)
    nb = pl.num_programs(0)

    @pl.when(i == 0)
    def _():
        acc_ref[...] = jnp.zeros_like(acc_ref)
        cnt_ref[...] = jnp.zeros_like(cnt_ref)

    t = (jnp.dot(h_ref[0], p1a_ref[...], preferred_element_type=jnp.float32)
         + jnp.dot(h_ref[1], p1b_ref[...], preferred_element_type=jnp.float32)
         + pb1_ref[...])
    t = jnp.maximum(t, 0.0)
    o = jnp.dot(t, p2_ref[...], preferred_element_type=jnp.float32) + pb2_ref[...]
    nrm = jnp.sqrt(jnp.sum(o * o, axis=1, keepdims=True))
    nf = o / (nrm + 1e-12)

    ids = bat_ref[...]
    gids = lax.broadcasted_iota(jnp.int32, (1, G), 1)
    oh = (ids == gids).astype(jnp.float32)
    acc_ref[...] += lax.dot_general(
        oh, nf, (((0,), (0,)), ((), ())), preferred_element_type=jnp.float32)
    ones = jnp.ones((ids.shape[0], 1), jnp.float32)
    cnt_ref[...] += lax.dot_general(
        oh, ones, (((0,), (0,)), ((), ())), preferred_element_type=jnp.float32)

    @pl.when(i == nb - 1)
    def _():
        gm = acc_ref[...] / jnp.maximum(cnt_ref[...], 1.0)
        gnrm = jnp.sqrt(jnp.sum(gm * gm, axis=1, keepdims=True))
        gf_ref[...] = gm / (gnrm + 1e-12)


def _proj_pool(h, p1a, p1b, pb1, p2, pb2, batc):
    nb = NPAD // _NBLK_ROWS
    row3 = lambda i: (0, i, 0)
    row = lambda i: (i, 0)
    rep = lambda i: (0, 0)
    return pl.pallas_call(
        _proj_pool_body,
        grid=(nb,),
        in_specs=[
            pl.BlockSpec((2, _NBLK_ROWS, HALF), row3),
            pl.BlockSpec((HALF, DP), rep),
            pl.BlockSpec((HALF, DP), rep),
            pl.BlockSpec((1, DP), rep),
            pl.BlockSpec((DP, DP), rep),
            pl.BlockSpec((1, DP), rep),
            pl.BlockSpec((_NBLK_ROWS, 1), row),
        ],
        out_specs=pl.BlockSpec((G, DP), rep),
        out_shape=jax.ShapeDtypeStruct((G, DP), jnp.float32),
        scratch_shapes=[
            pltpu.VMEM((G, DP), jnp.float32),
            pltpu.VMEM((G, 1), jnp.float32),
        ],
    )(h, p1a, p1b, pb1, p2, pb2, batc)


def _pad2(a, r, c):
    return jnp.pad(a, ((0, r - a.shape[0]), (0, c - a.shape[1])))


def kernel(x, edge_index, edge_attr, batch, We, be, W1, b1, W2, b2,
           P1, pb1, P2, pb2):
    f32 = jnp.float32
    x = x.astype(f32)

    src = edge_index[0].astype(jnp.int32).reshape(NTILES, NBLK, EB)
    dst = edge_index[1].astype(jnp.int32).reshape(NTILES, NBLK, EB)
    batc = jnp.concatenate(
        [batch.astype(jnp.int32), jnp.full((NPAD - N,), G, jnp.int32)]
    ).reshape(NPAD, 1)

    xp = _pad2(x, NPAD, DP)
    h = jnp.stack([xp[:, :HALF], xp[:, HALF:]])

    Wep = jnp.pad(We.astype(f32), ((0, 0), (0, 0), (0, DP - D)))
    bep = jnp.pad(be.astype(f32), ((0, 0), (0, DP - D)))
    w_e = Wep.reshape(L, DE, 2, HALF).transpose(0, 2, 1, 3)
    b_e = bep.reshape(L, 2, 1, HALF)
    perm = [g * 32 + (16 if i % 2 else 0) + i // 2
            for g in range(HALF // 32) for i in range(32)]
    w_e = w_e[..., perm]
    b_e = b_e[..., perm]

    W1p = jnp.pad(W1.astype(f32), ((0, 0), (0, DP - D), (0, HP - H)))
    b1p = jnp.pad(b1.astype(f32), ((0, 0), (0, HP - H)))
    W2p = jnp.pad(W2.astype(f32), ((0, 0), (0, HP - H), (0, DP - D)))
    b2p = jnp.pad(b2.astype(f32), ((0, 0), (0, DP - D)))
    P1p = _pad2(P1.astype(f32), DP, DP)
    pb1p = jnp.pad(pb1.astype(f32), (0, DP - D)).reshape(1, DP)
    P2p = _pad2(P2.astype(f32), DP, DP)
    pb2p = jnp.pad(pb2.astype(f32), (0, DP - D)).reshape(1, DP)

    zeros = jnp.zeros((_ROWS_PER_TILE, HALF), f32)

    e_ls = [_edge_encode(edge_attr.astype(f32), w_e[l], b_e[l])
            for l in range(L)]

    for l in range(L):
        a = _sc_message_agg(h, e_ls[l], src, dst, zeros)
        h = _mlp(
            l < L - 1, h, a,
            W1p[l, :HALF, :], W1p[l, HALF:, :], b1p[l].reshape(1, HP),
            W2p[l, :, :HALF], W2p[l, :, HALF:],
            b2p[l, :HALF].reshape(1, HALF), b2p[l, HALF:].reshape(1, HALF),
        )

    gf = _proj_pool(h, P1p[:HALF, :], P1p[HALF:, :], pb1p, P2p, pb2p, batc)
    return gf[:, :D]


# speedup vs baseline: 1.2754x; 1.2754x over previous
import functools

import jax
import jax.numpy as jnp
from jax import lax
from jax.experimental import pallas as pl
from jax.experimental.pallas import tpu as pltpu
from jax.experimental.pallas import tpu_sc as plsc

N = 10000
E = 160000
D = 300
DE = 16
H = 600
G = 256
L = 5

HALF = 160
DP = 2 * HALF
HP = 640
NPAD = 10240

EB = 40
NBLK = 250
_CHUNK = 25
_NCH = NBLK // _CHUNK
TILE_E = EB * NBLK
NTILES = 16
_ROWS_PER_TILE = N // NTILES


_EBLK = 640


def _edge_encode_body(ea_ref, w_ref, b_ref, out_ref):
    out_ref[...] = (
        jnp.dot(ea_ref[...], w_ref[...], preferred_element_type=jnp.float32)
        + b_ref[...]
    )


def _edge_encode(edge_attr, w, b):
    return pl.pallas_call(
        _edge_encode_body,
        grid=(E // _EBLK,),
        in_specs=[
            pl.BlockSpec((_EBLK, DE), lambda i: (i, 0)),
            pl.BlockSpec((DE, DP), lambda i: (0, 0)),
            pl.BlockSpec((1, DP), lambda i: (0, 0)),
        ],
        out_specs=pl.BlockSpec((_EBLK, DP), lambda i: (i, 0)),
        out_shape=jax.ShapeDtypeStruct((E, DP), jnp.float32),
    )(edge_attr, w, b)


def _sc_body(h_hbm, e_hbm, src_hbm, dst_hbm, zeros_hbm, agg_hbm,
             src_v, dst_v, h_a, h_b, e_a, e_b, agg_sh,
             sem_ha, sem_hb, sem_ea, sem_eb, sem_s):
    cid = lax.axis_index("c")
    sid = lax.axis_index("s")
    hsrc = h_hbm.at[cid]
    ecol = cid * HALF

    row0 = sid * _ROWS_PER_TILE
    pltpu.sync_copy(zeros_hbm, agg_sh.at[pl.ds(row0, _ROWS_PER_TILE)])
    plsc.subcore_barrier()

    def _issue(g, j, hbuf, ebuf, sem_h, sem_e):
        pltpu.async_copy(hsrc.at[src_v.at[j]], hbuf, sem_h)
        pltpu.async_copy(
            e_hbm.at[pl.ds(sid * TILE_E + g * EB, EB), pl.ds(ecol, HALF)],
            ebuf, sem_e)

    def _work(j, hbuf, ebuf, sem_h, sem_e):
        pltpu.make_async_copy(hsrc.at[src_v.at[j]], hbuf, sem_h).wait()
        pltpu.make_async_copy(
            e_hbm.at[pl.ds(0, EB), pl.ds(ecol, HALF)], ebuf, sem_e).wait()

        def _row(b, _):
            for k in range(HALF // 16):
                s = pl.ds(k * 16, 16)
                hbuf[b, s] = jnp.maximum(hbuf[b, s] + ebuf[b, s], 0.0)
            return 0

        lax.fori_loop(0, EB, _row, 0)
        pltpu.async_copy(hbuf, agg_sh.at[dst_v.at[j]], sem_s, add=True)

    def _chunk(c, _):
        @pl.when(c > 0)
        def _():
            pltpu.make_async_copy(h_a, agg_sh.at[dst_v.at[0]], sem_s).wait()

        pltpu.sync_copy(src_hbm.at[sid].at[pl.ds(c * _CHUNK, _CHUNK)], src_v)
        pltpu.sync_copy(dst_hbm.at[sid].at[pl.ds(c * _CHUNK, _CHUNK)], dst_v)

        @pl.when(c % 2 == 0)
        def _():
            _issue(c * _CHUNK, 0, h_a, e_a, sem_ha, sem_ea)

        @pl.when(c % 2 == 1)
        def _():
            _issue(c * _CHUNK, 0, h_b, e_b, sem_hb, sem_eb)

        def _blk(j, _):
            g = c * _CHUNK + j
            par = (c + j) % 2

            @pl.when(j > 0)
            def _():
                pltpu.make_async_copy(h_a, agg_sh.at[dst_v.at[0]],
                                      sem_s).wait()

            @pl.when(jnp.logical_and(j < _CHUNK - 1, par == 0))
            def _():
                _issue(g + 1, j + 1, h_b, e_b, sem_hb, sem_eb)

            @pl.when(jnp.logical_and(j < _CHUNK - 1, par == 1))
            def _():
                _issue(g + 1, j + 1, h_a, e_a, sem_ha, sem_ea)

            @pl.when(par == 0)
            def _():
                _work(j, h_a, e_a, sem_ha, sem_ea)

            @pl.when(par == 1)
            def _():
                _work(j, h_b, e_b, sem_hb, sem_eb)

            return 0

        lax.fori_loop(0, _CHUNK, _blk, 0)
        return 0

    lax.fori_loop(0, _NCH, _chunk, 0)
    pltpu.make_async_copy(h_a, agg_sh.at[dst_v.at[0]], sem_s).wait()
    plsc.subcore_barrier()

    pltpu.sync_copy(agg_sh.at[pl.ds(row0, _ROWS_PER_TILE)],
                    agg_hbm.at[cid].at[pl.ds(row0, _ROWS_PER_TILE)])

    @pl.when(sid == 0)
    def _():
        pltpu.sync_copy(zeros_hbm.at[pl.ds(0, NPAD - N)],
                        agg_hbm.at[cid].at[pl.ds(N, NPAD - N)])


def _sc_message_agg(h, e_l, srcr, dstr, zeros):
    mesh = plsc.VectorSubcoreMesh(core_axis_name="c", subcore_axis_name="s",
                                  num_cores=2, num_subcores=NTILES)
    k = pl.kernel(
        _sc_body,
        out_type=jax.ShapeDtypeStruct((2, NPAD, HALF), jnp.float32),
        mesh=mesh,
        scratch_types=[
            pltpu.VMEM((_CHUNK, EB), jnp.int32),
            pltpu.VMEM((_CHUNK, EB), jnp.int32),
            pltpu.VMEM((EB, HALF), jnp.float32),
            pltpu.VMEM((EB, HALF), jnp.float32),
            pltpu.VMEM((EB, HALF), jnp.float32),
            pltpu.VMEM((EB, HALF), jnp.float32),
            pltpu.VMEM_SHARED((N, HALF), jnp.float32),
            pltpu.SemaphoreType.DMA,
            pltpu.SemaphoreType.DMA,
            pltpu.SemaphoreType.DMA,
            pltpu.SemaphoreType.DMA,
            pltpu.SemaphoreType.DMA,
        ],
        compiler_params=pltpu.CompilerParams(use_tc_tiling_on_sc=False),
    )
    return k(h, e_l, srcr, dstr, zeros)


_NBLK_ROWS = 512


def _mlp_body(relu_out, h_ref, a_ref,
              w1a_ref, w1b_ref, b1_ref, w2a_ref, w2b_ref, b2a_ref, b2b_ref,
              o_ref):
    z0 = h_ref[0] + a_ref[0]
    z1 = h_ref[1] + a_ref[1]
    t = (jnp.dot(z0, w1a_ref[...], preferred_element_type=jnp.float32)
         + jnp.dot(z1, w1b_ref[...], preferred_element_type=jnp.float32)
         + b1_ref[...])
    t = jnp.maximum(t, 0.0)
    o0 = jnp.dot(t, w2a_ref[...], preferred_element_type=jnp.float32) + b2a_ref[...]
    o1 = jnp.dot(t, w2b_ref[...], preferred_element_type=jnp.float32) + b2b_ref[...]
    if relu_out:
        o0 = jnp.maximum(o0, 0.0)
        o1 = jnp.maximum(o1, 0.0)
    o_ref[0] = o0
    o_ref[1] = o1


def _mlp(relu_out, h, a, w1a, w1b, b1, w2a, w2b, b2a, b2b):
    nb = NPAD // _NBLK_ROWS
    row = lambda i: (0, i, 0)
    rep = lambda i: (0, 0)
    return pl.pallas_call(
        functools.partial(_mlp_body, relu_out),
        grid=(nb,),
        in_specs=[
            pl.BlockSpec((2, _NBLK_ROWS, HALF), row),
            pl.BlockSpec((2, _NBLK_ROWS, HALF), row),
            pl.BlockSpec((HALF, HP), rep),
            pl.BlockSpec((HALF, HP), rep),
            pl.BlockSpec((1, HP), rep),
            pl.BlockSpec((HP, HALF), rep),
            pl.BlockSpec((HP, HALF), rep),
            pl.BlockSpec((1, HALF), rep),
            pl.BlockSpec((1, HALF), rep),
        ],
        out_specs=pl.BlockSpec((2, _NBLK_ROWS, HALF), row),
        out_shape=jax.ShapeDtypeStruct((2, NPAD, HALF), jnp.float32),
    )(h, a, w1a, w1b, b1, w2a, w2b, b2a, b2b)


def _proj_pool_body(h_ref, p1a_ref, p1b_ref, pb1_ref, p2_ref,
                    pb2_ref, bat_ref, gf_ref, acc_ref, cnt_ref):
    i = pl.program_id(0)
    nb = pl.num_programs(0)

    @pl.when(i == 0)
    def _():
        acc_ref[...] = jnp.zeros_like(acc_ref)
        cnt_ref[...] = jnp.zeros_like(cnt_ref)

    t = (jnp.dot(h_ref[0], p1a_ref[...], preferred_element_type=jnp.float32)
         + jnp.dot(h_ref[1], p1b_ref[...], preferred_element_type=jnp.float32)
         + pb1_ref[...])
    t = jnp.maximum(t, 0.0)
    o = jnp.dot(t, p2_ref[...], preferred_element_type=jnp.float32) + pb2_ref[...]
    nrm = jnp.sqrt(jnp.sum(o * o, axis=1, keepdims=True))
    nf = o / (nrm + 1e-12)

    ids = bat_ref[...]
    gids = lax.broadcasted_iota(jnp.int32, (1, G), 1)
    oh = (ids == gids).astype(jnp.float32)
    acc_ref[...] += lax.dot_general(
        oh, nf, (((0,), (0,)), ((), ())), preferred_element_type=jnp.float32)
    ones = jnp.ones((ids.shape[0], 1), jnp.float32)
    cnt_ref[...] += lax.dot_general(
        oh, ones, (((0,), (0,)), ((), ())), preferred_element_type=jnp.float32)

    @pl.when(i == nb - 1)
    def _():
        gm = acc_ref[...] / jnp.maximum(cnt_ref[...], 1.0)
        gnrm = jnp.sqrt(jnp.sum(gm * gm, axis=1, keepdims=True))
        gf_ref[...] = gm / (gnrm + 1e-12)


def _proj_pool(h, p1a, p1b, pb1, p2, pb2, batc):
    nb = NPAD // _NBLK_ROWS
    row3 = lambda i: (0, i, 0)
    row = lambda i: (i, 0)
    rep = lambda i: (0, 0)
    return pl.pallas_call(
        _proj_pool_body,
        grid=(nb,),
        in_specs=[
            pl.BlockSpec((2, _NBLK_ROWS, HALF), row3),
            pl.BlockSpec((HALF, DP), rep),
            pl.BlockSpec((HALF, DP), rep),
            pl.BlockSpec((1, DP), rep),
            pl.BlockSpec((DP, DP), rep),
            pl.BlockSpec((1, DP), rep),
            pl.BlockSpec((_NBLK_ROWS, 1), row),
        ],
        out_specs=pl.BlockSpec((G, DP), rep),
        out_shape=jax.ShapeDtypeStruct((G, DP), jnp.float32),
        scratch_shapes=[
            pltpu.VMEM((G, DP), jnp.float32),
            pltpu.VMEM((G, 1), jnp.float32),
        ],
    )(h, p1a, p1b, pb1, p2, pb2, batc)


def _pad2(a, r, c):
    return jnp.pad(a, ((0, r - a.shape[0]), (0, c - a.shape[1])))


def kernel(x, edge_index, edge_attr, batch, We, be, W1, b1, W2, b2,
           P1, pb1, P2, pb2):
    f32 = jnp.float32
    x = x.astype(f32)

    src = edge_index[0].astype(jnp.int32).reshape(NTILES, NBLK, EB)
    dst = edge_index[1].astype(jnp.int32).reshape(NTILES, NBLK, EB)
    batc = jnp.concatenate(
        [batch.astype(jnp.int32), jnp.full((NPAD - N,), G, jnp.int32)]
    ).reshape(NPAD, 1)

    xp = _pad2(x, NPAD, DP)
    h = jnp.stack([xp[:, :HALF], xp[:, HALF:]])

    Wep = jnp.pad(We.astype(f32), ((0, 0), (0, 0), (0, DP - D)))
    bep = jnp.pad(be.astype(f32), ((0, 0), (0, DP - D)))

    W1p = jnp.pad(W1.astype(f32), ((0, 0), (0, DP - D), (0, HP - H)))
    b1p = jnp.pad(b1.astype(f32), ((0, 0), (0, HP - H)))
    W2p = jnp.pad(W2.astype(f32), ((0, 0), (0, HP - H), (0, DP - D)))
    b2p = jnp.pad(b2.astype(f32), ((0, 0), (0, DP - D)))
    P1p = _pad2(P1.astype(f32), DP, DP)
    pb1p = jnp.pad(pb1.astype(f32), (0, DP - D)).reshape(1, DP)
    P2p = _pad2(P2.astype(f32), DP, DP)
    pb2p = jnp.pad(pb2.astype(f32), (0, DP - D)).reshape(1, DP)

    zeros = jnp.zeros((_ROWS_PER_TILE, HALF), f32)

    e_ls = [_edge_encode(edge_attr.astype(f32), Wep[l], bep[l].reshape(1, DP))
            for l in range(L)]

    for l in range(L):
        a = _sc_message_agg(h, e_ls[l], src, dst, zeros)
        h = _mlp(
            l < L - 1, h, a,
            W1p[l, :HALF, :], W1p[l, HALF:, :], b1p[l].reshape(1, HP),
            W2p[l, :, :HALF], W2p[l, :, HALF:],
            b2p[l, :HALF].reshape(1, HALF), b2p[l, HALF:].reshape(1, HALF),
        )

    gf = _proj_pool(h, P1p[:HALF, :], P1p[HALF:, :], pb1p, P2p, pb2p, batc)
    return gf[:, :D]
